# baseline (speedup 1.0000x reference)
import functools

import jax
import jax.numpy as jnp
from jax import lax
from jax.experimental import pallas as pl
from jax.experimental.pallas import tpu as pltpu
from jax.experimental.pallas import tpu_sc as plsc

N = 50000
E = 800000
H = 64
TW = 96
BN = 1000
NBN = N // BN

SC_CORES = 2
SC_SUB = 16
HALF = N // SC_CORES
K = 48
NB = 1042
NBH = NB // 2
EPS = K * NB
EPAD = SC_SUB * EPS
RROWS = 25088
ZQ = RROWS // SC_SUB
CQ = 1560


def _relu(v):
    return jnp.maximum(v, 0.0)


def _embed_tables_body(nf, we, be, w1, b1, aw, ab, xo, ts, td):
    x = _relu(jnp.dot(nf[...], we[...], preferred_element_type=jnp.float32)
              + be[...])
    xo[...] = x
    ts[...] = jnp.concatenate(
        [jnp.dot(x, w1[:H], preferred_element_type=jnp.float32),
         jnp.dot(x, aw[:H], preferred_element_type=jnp.float32)], axis=1)
    td[...] = jnp.concatenate(
        [jnp.dot(x, w1[H:], preferred_element_type=jnp.float32) + b1[...],
         jnp.dot(x, aw[H:], preferred_element_type=jnp.float32) + ab[...]],
        axis=1)


def _embed_tables(nf, we, be, w1, b1, aw, ab):
    blk = lambda r, c: pl.BlockSpec((r, c), lambda i: (i, 0))
    rep = lambda r, c: pl.BlockSpec((r, c), lambda i: (0, 0))
    return pl.pallas_call(
        _embed_tables_body,
        grid=(NBN,),
        in_specs=[blk(BN, 18), rep(18, H), rep(1, H), rep(2 * H, H),
                  rep(1, H), rep(2 * H, H // 2), rep(1, H // 2)],
        out_specs=[blk(BN, H), blk(BN, TW), blk(BN, TW)],
        out_shape=[jax.ShapeDtypeStruct((N, H), jnp.float32),
                   jax.ShapeDtypeStruct((N, TW), jnp.float32),
                   jax.ShapeDtypeStruct((N, TW), jnp.float32)],
    )(nf, we, be, w1, b1, aw, ab)


def _post_agg(x, r, s, w2, b2, u1, c1, u2, c2):
    agg = (jnp.dot(r, w2[...], preferred_element_type=jnp.float32)
           + s * b2[...]) / jnp.maximum(s, 1e-6)
    xn = _relu(jnp.dot(x, u1[:H], preferred_element_type=jnp.float32)
               + jnp.dot(agg, u1[H:], preferred_element_type=jnp.float32)
               + c1[...])
    xn = jnp.dot(xn, u2[...], preferred_element_type=jnp.float32) + c2[...]
    return _relu(xn + x)


def _post_tables_body(xr, Rr, sr, w2, b2, u1, c1, u2, c2, w1, b1, aw, ab,
                      xo, ts, td):
    x = _post_agg(xr[...], Rr[...], sr[...], w2, b2, u1, c1, u2, c2)
    xo[...] = x
    ts[...] = jnp.concatenate(
        [jnp.dot(x, w1[:H], preferred_element_type=jnp.float32),
         jnp.dot(x, aw[:H], preferred_element_type=jnp.float32)], axis=1)
    td[...] = jnp.concatenate(
        [jnp.dot(x, w1[H:], preferred_element_type=jnp.float32) + b1[...],
         jnp.dot(x, aw[H:], preferred_element_type=jnp.float32) + ab[...]],
        axis=1)


def _post_tables(x, R, s, w2, b2, u1, c1, u2, c2, w1, b1, aw, ab):
    blk = lambda r, c: pl.BlockSpec((r, c), lambda i: (i, 0))
    rep = lambda r, c: pl.BlockSpec((r, c), lambda i: (0, 0))
    return pl.pallas_call(
        _post_tables_body,
        grid=(NBN,),
        in_specs=[blk(BN, H), blk(BN, H), blk(BN, 1), rep(H, H), rep(1, H),
                  rep(2 * H, H), rep(1, H), rep(H, H), rep(1, H),
                  rep(2 * H, H), rep(1, H), rep(2 * H, H // 2),
                  rep(1, H // 2)],
        out_specs=[blk(BN, H), blk(BN, TW), blk(BN, TW)],
        out_shape=[jax.ShapeDtypeStruct((N, H), jnp.float32),
                   jax.ShapeDtypeStruct((N, TW), jnp.float32),
                   jax.ShapeDtypeStruct((N, TW), jnp.float32)],
    )(x, R, s, w2, b2, u1, c1, u2, c2, w1, b1, aw, ab)


def _post_head_body(xr, Rr, sr, w2, b2, u1, c1, u2, c2, hw, hb, acc, out):
    x = _post_agg(xr[...], Rr[...], sr[...], w2, b2, u1, c1, u2, c2)
    m = jnp.max(x, axis=0, keepdims=True)
    i = pl.program_id(0)

    @pl.when(i == 0)
    def _():
        acc[...] = m

    @pl.when(i > 0)
    def _():
        acc[...] = jnp.maximum(acc[...], m)

    @pl.when(i == NBN - 1)
    def _():
        out[...] = (jnp.dot(acc[...], hw[...],
                            preferred_element_type=jnp.float32)
                    + hb[...])


def _post_head(x, R, s, w2, b2, u1, c1, u2, c2, hw, hb):
    blk = lambda r, c: pl.BlockSpec((r, c), lambda i: (i, 0))
    rep = lambda r, c: pl.BlockSpec((r, c), lambda i: (0, 0))
    acc, out = pl.pallas_call(
        _post_head_body,
        grid=(NBN,),
        in_specs=[blk(BN, H), blk(BN, H), blk(BN, 1), rep(H, H), rep(1, H),
                  rep(2 * H, H), rep(1, H), rep(H, H), rep(1, H),
                  rep(H, 1), rep(1, 1)],
        out_specs=[rep(1, H), rep(1, 1)],
        out_shape=[jax.ShapeDtypeStruct((1, H), jnp.float32),
                   jax.ShapeDtypeStruct((1, 1), jnp.float32)],
    )(x, R, s, w2, b2, u1, c1, u2, c2, hw, hb)
    return out


_SC_MESH = plsc.VectorSubcoreMesh(core_axis_name="c", subcore_axis_name="s")


@functools.partial(
    pl.kernel,
    mesh=_SC_MESH,
    compiler_params=pltpu.CompilerParams(use_tc_tiling_on_sc=False),
    out_type=[jax.ShapeDtypeStruct((N, H), jnp.float32),
              jax.ShapeDtypeStruct((N,), jnp.float32)],
    scratch_types=[
        pltpu.VMEM((2, K), jnp.int32),
        pltpu.VMEM((2, K), jnp.int32),
        pltpu.VMEM((2, K), jnp.int32),
        pltpu.VMEM((2, K), jnp.float32),
        pltpu.VMEM((2, K, TW), jnp.float32),
        pltpu.VMEM((2, K, TW), jnp.float32),
        pltpu.VMEM((2, K, H), jnp.float32),
        pltpu.VMEM((2, K), jnp.float32),
        pltpu.VMEM((16, 48), jnp.float32),
        pltpu.VMEM((48,), jnp.float32),
        pltpu.VMEM_SHARED((RROWS, H), jnp.float32),
        pltpu.VMEM_SHARED((RROWS,), jnp.float32),
        pltpu.SemaphoreType.DMA,
        pltpu.SemaphoreType.DMA,
        pltpu.SemaphoreType.DMA,
        pltpu.SemaphoreType.DMA,
        pltpu.SemaphoreType.DMA,
        pltpu.SemaphoreType.DMA,
    ],
)
def _sc_edges(ts_hbm, td_hbm, row_hbm, col_hbm, ew_hbm, ap_hbm,
              r_hbm, f_hbm,
              row_v, col_v, sidx_v, ew_v, src_v, dst_v, out_v, fw1_v,
              pad_v, ap_v, racc, facc,
              sem_i0, sem_i1, sem_g0, sem_g1, sem_s0, sem_s1):
    c = lax.axis_index("c")
    s = lax.axis_index("s")
    zero16 = jnp.zeros((16,), jnp.float32)
    iota16 = lax.iota(jnp.int32, 16)
    sem_i = (sem_i0, sem_i1)
    sem_g = (sem_g0, sem_g1)
    sem_s = (sem_s0, sem_s1)
    sbase = s * EPS
    noff = c * HALF

    def _issue_idx(j, b):
        base = sbase + j * K
        pltpu.async_copy(row_hbm.at[pl.ds(base, K)], row_v.at[b], sem_i[b])
        pltpu.async_copy(col_hbm.at[pl.ds(base, K)], col_v.at[b], sem_i[b])
        pltpu.async_copy(ew_hbm.at[pl.ds(base, K)], ew_v.at[b], sem_i[b])

    def _drain_idx(b):
        pltpu.make_async_copy(row_hbm.at[pl.ds(0, K)], row_v.at[b],
                              sem_i[b]).wait()
        pltpu.make_async_copy(col_hbm.at[pl.ds(0, K)], col_v.at[b],
                              sem_i[b]).wait()
        pltpu.make_async_copy(ew_hbm.at[pl.ds(0, K)], ew_v.at[b],
                              sem_i[b]).wait()

    def _issue_gather(b):
        pltpu.async_copy(ts_hbm.at[row_v.at[b]], src_v.at[b], sem_g[b])
        pltpu.async_copy(td_hbm.at[col_v.at[b]], dst_v.at[b], sem_g[b])

    def _drain_gather(b):
        pltpu.make_async_copy(ts_hbm.at[row_v.at[b]], src_v.at[b],
                              sem_g[b]).wait()
        pltpu.make_async_copy(td_hbm.at[col_v.at[b]], dst_v.at[b],
                              sem_g[b]).wait()

    def _issue_scatter(b):
        pltpu.async_copy(out_v.at[b], racc.at[sidx_v.at[b]], sem_s[b],
                         add=True)
        pltpu.async_copy(fw1_v.at[b], facc.at[sidx_v.at[b]], sem_s[b],
                         add=True)

    def _drain_scatter(b):
        pltpu.make_async_copy(r_hbm.at[pl.ds(0, K)], out_v.at[b],
                              sem_s[b]).wait()
        pltpu.make_async_copy(f_hbm.at[pl.ds(0, K)], fw1_v.at[b],
                              sem_s[b]).wait()

    _issue_idx(0, 0)
    _issue_idx(1, 1)
    _drain_idx(0)
    _issue_gather(0)

    def _zrow(e, _):
        for b in range(2):
            for d in range(H // 16):
                out_v[b, e, pl.ds(16 * d, 16)] = zero16
        return 0
    lax.fori_loop(0, K, _zrow, 0)
    for b in range(2):
        for g in range(K // 16):
            fw1_v[b, pl.ds(16 * g, 16)] = zero16
    for el in range(16):
        pad_v[el, pl.ds(0, 16)] = zero16
        pad_v[el, pl.ds(32, 16)] = zero16

    zbase = s * ZQ

    def _zcp(k, _):
        pltpu.sync_copy(out_v.at[0], racc.at[pl.ds(zbase + k * K, K)])
        pltpu.sync_copy(fw1_v.at[0], facc.at[pl.ds(zbase + k * K, K)])
        return 0
    lax.fori_loop(0, ZQ // K, _zcp, 0)
    pltpu.sync_copy(out_v.at[0].at[pl.ds(0, ZQ % K)],
                    racc.at[pl.ds(zbase + (ZQ // K) * K, ZQ % K)])
    pltpu.sync_copy(fw1_v.at[0].at[pl.ds(0, ZQ % K)],
                    facc.at[pl.ds(zbase + (ZQ // K) * K, ZQ % K)])

    pltpu.sync_copy(ap_hbm, ap_v)
    plsc.subcore_barrier()

    aw0 = ap_v[pl.ds(0, 16)]
    aw1 = ap_v[pl.ds(16, 16)]
    a2v = ap_v[pl.ds(32, 16)]

    def _compute(b):
        def _group(g, _):
            gb = g * 16
            cv = col_v[b, pl.ds(gb, 16)]
            loc = cv - noff
            inr = (loc >= 0) & (loc < HALF)
            sidx_v[b, pl.ds(gb, 16)] = jnp.where(inr, loc, HALF)

            avec = zero16
            for el in range(16):
                e = gb + el
                t4 = jnp.maximum(src_v[b, e, pl.ds(64, 16)]
                                 + dst_v[b, e, pl.ds(64, 16)], 0.0)
                t5 = jnp.maximum(src_v[b, e, pl.ds(80, 16)]
                                 + dst_v[b, e, pl.ds(80, 16)], 0.0)
                w = t4 * aw0 + t5 * aw1
                for sh in (8, 4, 2, 1):
                    pad_v[el, pl.ds(16, 16)] = w
                    w = w + pad_v[el, pl.ds(16 + sh, 16)]
                pad_v[el, pl.ds(16, 16)] = w
                v = pad_v[el, pl.ds(16 - el, 16)]
                avec = avec + jnp.where(iota16 == el, v, 0.0)
            sg = 1.0 / (1.0 + jnp.exp(-(avec + a2v)))
            fw = ew_v[b, pl.ds(gb, 16)] * sg
            fw1_v[b, pl.ds(gb, 16)] = fw

            for el in range(16):
                e = gb + el
                fws = lax.squeeze(lax.slice(fw, (el,), (el + 1,)), (0,))
                fwb = jnp.full((16,), fws, jnp.float32)
                for d in range(H // 16):
                    sv = src_v[b, e, pl.ds(16 * d, 16)]
                    dv = dst_v[b, e, pl.ds(16 * d, 16)]
                    out_v[b, e, pl.ds(16 * d, 16)] = (
                        jnp.maximum(sv + dv, 0.0) * fwb)
            return 0

        lax.fori_loop(0, K // 16, _group, 0)

    def _iter(i, _):
        for b in range(2):
            j = 2 * i + b

            @pl.when(i >= 1)
            def _():
                _drain_scatter(b)

            if b == 0:
                _drain_idx(1)
                _issue_gather(1)
            else:
                @pl.when(i < NBH - 1)
                def _():
                    _drain_idx(0)
                    _issue_gather(0)

            _drain_gather(b)
            _compute(b)
            _issue_scatter(b)

            @pl.when(i < NBH - 1)
            def _():
                _issue_idx(j + 2, b)
        return 0

    lax.fori_loop(0, NBH, _iter, 0)
    _drain_scatter(0)
    _drain_scatter(1)
    plsc.subcore_barrier()

    cbase = s * CQ

    def _cp(k, _):
        pltpu.sync_copy(racc.at[pl.ds(cbase + k * 40, 40)],
                        r_hbm.at[pl.ds(noff + cbase + k * 40, 40)])
        return 0
    lax.fori_loop(0, CQ // 40, _cp, 0)
    pltpu.sync_copy(facc.at[pl.ds(cbase, CQ)],
                    f_hbm.at[pl.ds(noff + cbase, CQ)])

    @pl.when(s == SC_SUB - 1)
    def _():
        pltpu.sync_copy(racc.at[pl.ds(cbase + CQ, 40)],
                        r_hbm.at[pl.ds(noff + cbase + CQ, 40)])
        pltpu.sync_copy(facc.at[pl.ds(cbase + CQ, 40)],
                        f_hbm.at[pl.ds(noff + cbase + CQ, 40)])


def kernel(node_features, edge_index, edge_weights, params):
    p = params
    padi = jnp.zeros((EPAD - E,), jnp.int32)
    row = jnp.concatenate([edge_index[0], padi])
    col = jnp.concatenate([edge_index[1], padi])
    ew = jnp.concatenate([edge_weights, jnp.zeros((EPAD - E,), jnp.float32)])
    r2 = lambda b: b.reshape(1, -1)

    ap = [jnp.concatenate([p['l%d_att2_w' % i][:, 0],
                           jnp.full((16,), p['l%d_att2_b' % i][0],
                                    jnp.float32)]) for i in range(2)]

    x0, ts0, td0 = _embed_tables(
        node_features, p['embed_w'], r2(p['embed_b']),
        p['l0_msg1_w'], r2(p['l0_msg1_b']),
        p['l0_att1_w'], r2(p['l0_att1_b']))
    R0, f0 = _sc_edges(ts0, td0, row, col, ew, ap[0])
    x1, ts1, td1 = _post_tables(
        x0, R0, f0.reshape(N, 1), p['l0_msg2_w'], r2(p['l0_msg2_b']),
        p['l0_upd1_w'], r2(p['l0_upd1_b']),
        p['l0_upd2_w'], r2(p['l0_upd2_b']),
        p['l1_msg1_w'], r2(p['l1_msg1_b']),
        p['l1_att1_w'], r2(p['l1_att1_b']))
    R1, f1 = _sc_edges(ts1, td1, row, col, ew, ap[1])
    out = _post_head(
        x1, R1, f1.reshape(N, 1), p['l1_msg2_w'], r2(p['l1_msg2_b']),
        p['l1_upd1_w'], r2(p['l1_upd1_b']),
        p['l1_upd2_w'], r2(p['l1_upd2_b']),
        p['head_w'], r2(p['head_b']))
    return out[0, 0]


# speedup vs baseline: 1.4102x; 1.4102x over previous
import functools

import jax
import jax.numpy as jnp
from jax import lax
from jax.experimental import pallas as pl
from jax.experimental.pallas import tpu as pltpu
from jax.experimental.pallas import tpu_sc as plsc

N = 50000
E = 800000
H = 64
HH = 32
TW = 64
BN = 1000
NBN = N // BN

SC_CORES = 2
SC_SUB = 16
K = 80
NB = 626
NBH = NB // 2
EPS = K * NB
EPAD = SC_SUB * EPS
RROWS = 50048
ZQ = RROWS // SC_SUB
CQ = 3120


def _relu(v):
    return jnp.maximum(v, 0.0)


def _tables(x, w1, b1, aw, ab):
    m = jnp.dot(x, w1, preferred_element_type=jnp.float32)
    a = jnp.dot(x, aw, preferred_element_type=jnp.float32)
    if b1 is not None:
        m = m + b1
        a = a + ab
    ta = jnp.concatenate([m[:, :HH], a], axis=1)
    tb = jnp.concatenate([m[:, HH:], a], axis=1)
    return ta, tb


def _embed_tables_body(nf, we, be, w1, b1, aw, ab, xo, tsa, tsb, tda, tdb):
    x = _relu(jnp.dot(nf[...], we[...], preferred_element_type=jnp.float32)
              + be[...])
    xo[...] = x
    tsa[...], tsb[...] = _tables(x, w1[:H], None, aw[:H], None)
    tda[...], tdb[...] = _tables(x, w1[H:], b1[...], aw[H:], ab[...])


def _embed_tables(nf, we, be, w1, b1, aw, ab):
    blk = lambda r, c: pl.BlockSpec((r, c), lambda i: (i, 0))
    rep = lambda r, c: pl.BlockSpec((r, c), lambda i: (0, 0))
    t64 = jax.ShapeDtypeStruct((N, TW), jnp.float32)
    return pl.pallas_call(
        _embed_tables_body,
        grid=(NBN,),
        in_specs=[blk(BN, 18), rep(18, H), rep(1, H), rep(2 * H, H),
                  rep(1, H), rep(2 * H, H // 2), rep(1, H // 2)],
        out_specs=[blk(BN, H), blk(BN, TW), blk(BN, TW), blk(BN, TW),
                   blk(BN, TW)],
        out_shape=[jax.ShapeDtypeStruct((N, H), jnp.float32),
                   t64, t64, t64, t64],
    )(nf, we, be, w1, b1, aw, ab)


def _post_agg(x, r0, r1, s, w2, b2, u1, c1, u2, c2):
    num = (jnp.dot(r0, w2[:HH], preferred_element_type=jnp.float32)
           + jnp.dot(r1, w2[HH:], preferred_element_type=jnp.float32)
           + s * b2[...])
    agg = num / jnp.maximum(s, 1e-6)
    xn = _relu(jnp.dot(x, u1[:H], preferred_element_type=jnp.float32)
               + jnp.dot(agg, u1[H:], preferred_element_type=jnp.float32)
               + c1[...])
    xn = jnp.dot(xn, u2[...], preferred_element_type=jnp.float32) + c2[...]
    return _relu(xn + x)


def _post_tables_body(xr, r0, r1, sr, w2, b2, u1, c1, u2, c2, w1, b1,
                      aw, ab, xo, tsa, tsb, tda, tdb):
    x = _post_agg(xr[...], r0[...], r1[...], sr[...], w2, b2, u1, c1,
                  u2, c2)
    xo[...] = x
    tsa[...], tsb[...] = _tables(x, w1[:H], None, aw[:H], None)
    tda[...], tdb[...] = _tables(x, w1[H:], b1[...], aw[H:], ab[...])


def _post_tables(x, r0, r1, s, w2, b2, u1, c1, u2, c2, w1, b1, aw, ab):
    blk = lambda r, c: pl.BlockSpec((r, c), lambda i: (i, 0))
    rep = lambda r, c: pl.BlockSpec((r, c), lambda i: (0, 0))
    t64 = jax.ShapeDtypeStruct((N, TW), jnp.float32)
    return pl.pallas_call(
        _post_tables_body,
        grid=(NBN,),
        in_specs=[blk(BN, H), blk(BN, HH), blk(BN, HH), blk(BN, 1),
                  rep(H, H), rep(1, H),
                  rep(2 * H, H), rep(1, H), rep(H, H), rep(1, H),
                  rep(2 * H, H), rep(1, H), rep(2 * H, H // 2),
                  rep(1, H // 2)],
        out_specs=[blk(BN, H), blk(BN, TW), blk(BN, TW), blk(BN, TW),
                   blk(BN, TW)],
        out_shape=[jax.ShapeDtypeStruct((N, H), jnp.float32),
                   t64, t64, t64, t64],
    )(x, r0, r1, s, w2, b2, u1, c1, u2, c2, w1, b1, aw, ab)


def _post_head_body(xr, r0, r1, sr, w2, b2, u1, c1, u2, c2, hw, hb,
                    acc, out):
    x = _post_agg(xr[...], r0[...], r1[...], sr[...], w2, b2, u1, c1,
                  u2, c2)
    m = jnp.max(x, axis=0, keepdims=True)
    i = pl.program_id(0)

    @pl.when(i == 0)
    def _():
        acc[...] = m

    @pl.when(i > 0)
    def _():
        acc[...] = jnp.maximum(acc[...], m)

    @pl.when(i == NBN - 1)
    def _():
        out[...] = (jnp.dot(acc[...], hw[...],
                            preferred_element_type=jnp.float32)
                    + hb[...])


def _post_head(x, r0, r1, s, w2, b2, u1, c1, u2, c2, hw, hb):
    blk = lambda r, c: pl.BlockSpec((r, c), lambda i: (i, 0))
    rep = lambda r, c: pl.BlockSpec((r, c), lambda i: (0, 0))
    acc, out = pl.pallas_call(
        _post_head_body,
        grid=(NBN,),
        in_specs=[blk(BN, H), blk(BN, HH), blk(BN, HH), blk(BN, 1),
                  rep(H, H), rep(1, H),
                  rep(2 * H, H), rep(1, H), rep(H, H), rep(1, H),
                  rep(H, 1), rep(1, 1)],
        out_specs=[rep(1, H), rep(1, 1)],
        out_shape=[jax.ShapeDtypeStruct((1, H), jnp.float32),
                   jax.ShapeDtypeStruct((1, 1), jnp.float32)],
    )(x, r0, r1, s, w2, b2, u1, c1, u2, c2, hw, hb)
    return out


_SC_MESH = plsc.VectorSubcoreMesh(core_axis_name="c", subcore_axis_name="s")


@functools.partial(
    pl.kernel,
    mesh=_SC_MESH,
    compiler_params=pltpu.CompilerParams(use_tc_tiling_on_sc=False),
    out_type=[jax.ShapeDtypeStruct((N, HH), jnp.float32),
              jax.ShapeDtypeStruct((N, HH), jnp.float32),
              jax.ShapeDtypeStruct((N,), jnp.float32)],
    scratch_types=[
        pltpu.VMEM((2, K), jnp.int32),
        pltpu.VMEM((2, K), jnp.int32),
        pltpu.VMEM((2, K), jnp.int32),
        pltpu.VMEM((2, K), jnp.float32),
        pltpu.VMEM((2, K, TW), jnp.float32),
        pltpu.VMEM((2, K, TW), jnp.float32),
        pltpu.VMEM((2, K, HH), jnp.float32),
        pltpu.VMEM((2, K), jnp.float32),
        pltpu.VMEM((16, 32), jnp.float32),
        pltpu.VMEM((48,), jnp.float32),
        pltpu.VMEM_SHARED((RROWS, HH), jnp.float32),
        pltpu.VMEM_SHARED((RROWS,), jnp.float32),
        pltpu.SemaphoreType.DMA,
        pltpu.SemaphoreType.DMA,
        pltpu.SemaphoreType.DMA,
        pltpu.SemaphoreType.DMA,
        pltpu.SemaphoreType.DMA,
        pltpu.SemaphoreType.DMA,
    ],
)
def _sc_edges(ts_hbm, td_hbm, row_hbm, col_hbm, ew_hbm, ap_hbm,
              r0_hbm, r1_hbm, f_hbm,
              row_v, col_v, cix_v, ew_v, src_v, dst_v, out_v, fw1_v,
              pad_v, ap_v, racc, facc,
              sem_i0, sem_i1, sem_g0, sem_g1, sem_s0, sem_s1):
    c = lax.axis_index("c")
    s = lax.axis_index("s")
    zero16 = jnp.zeros((16,), jnp.float32)
    iota16 = lax.iota(jnp.int32, 16)
    sem_i = (sem_i0, sem_i1)
    sem_g = (sem_g0, sem_g1)
    sem_s = (sem_s0, sem_s1)
    sbase = s * EPS
    coff = c * N

    def _issue_idx(j, b):
        base = sbase + j * K
        pltpu.async_copy(row_hbm.at[pl.ds(base, K)], row_v.at[b], sem_i[b])
        pltpu.async_copy(col_hbm.at[pl.ds(base, K)], col_v.at[b], sem_i[b])
        pltpu.async_copy(ew_hbm.at[pl.ds(base, K)], ew_v.at[b], sem_i[b])

    def _drain_idx(b):
        pltpu.make_async_copy(row_hbm.at[pl.ds(0, K)], row_v.at[b],
                              sem_i[b]).wait()
        pltpu.make_async_copy(col_hbm.at[pl.ds(0, K)], col_v.at[b],
                              sem_i[b]).wait()
        pltpu.make_async_copy(ew_hbm.at[pl.ds(0, K)], ew_v.at[b],
                              sem_i[b]).wait()

    def _adj_idx(b):
        for g in range(K // 16):
            sl = pl.ds(g * 16, 16)
            row_v[b, sl] = row_v[b, sl] + coff
            cix_v[b, sl] = col_v[b, sl] + coff

    def _issue_gather(b):
        pltpu.async_copy(ts_hbm.at[row_v.at[b]], src_v.at[b], sem_g[b])
        pltpu.async_copy(td_hbm.at[cix_v.at[b]], dst_v.at[b], sem_g[b])

    def _drain_gather(b):
        pltpu.make_async_copy(ts_hbm.at[row_v.at[b]], src_v.at[b],
                              sem_g[b]).wait()
        pltpu.make_async_copy(td_hbm.at[cix_v.at[b]], dst_v.at[b],
                              sem_g[b]).wait()

    def _issue_scatter(b):
        pltpu.async_copy(out_v.at[b], racc.at[col_v.at[b]], sem_s[b],
                         add=True)

        @pl.when(c == 0)
        def _():
            pltpu.async_copy(fw1_v.at[b], facc.at[col_v.at[b]], sem_s[b],
                             add=True)

    def _drain_scatter(b):
        pltpu.make_async_copy(r0_hbm.at[pl.ds(0, K)], out_v.at[b],
                              sem_s[b]).wait()

        @pl.when(c == 0)
        def _():
            pltpu.make_async_copy(f_hbm.at[pl.ds(0, K)], fw1_v.at[b],
                                  sem_s[b]).wait()

    _issue_idx(0, 0)
    _issue_idx(1, 1)
    _drain_idx(0)
    _adj_idx(0)
    _issue_gather(0)

    def _zrow(e, _):
        for b in range(2):
            for d in range(HH // 16):
                out_v[b, e, pl.ds(16 * d, 16)] = zero16
        return 0
    lax.fori_loop(0, K, _zrow, 0)
    for b in range(2):
        for g in range(K // 16):
            fw1_v[b, pl.ds(16 * g, 16)] = zero16
    for el in range(16):
        pad_v[el, pl.ds(16, 16)] = zero16

    zbase = s * ZQ

    def _zcp(k, _):
        pltpu.sync_copy(out_v.at[0], racc.at[pl.ds(zbase + k * K, K)])
        pltpu.sync_copy(fw1_v.at[0], facc.at[pl.ds(zbase + k * K, K)])
        return 0
    lax.fori_loop(0, ZQ // K, _zcp, 0)
    pltpu.sync_copy(out_v.at[0].at[pl.ds(0, ZQ % K)],
                    racc.at[pl.ds(zbase + (ZQ // K) * K, ZQ % K)])
    pltpu.sync_copy(fw1_v.at[0].at[pl.ds(0, ZQ % K)],
                    facc.at[pl.ds(zbase + (ZQ // K) * K, ZQ % K)])

    pltpu.sync_copy(ap_hbm, ap_v)
    plsc.subcore_barrier()

    aw0 = ap_v[pl.ds(0, 16)]
    aw1 = ap_v[pl.ds(16, 16)]
    a2v = ap_v[pl.ds(32, 16)]

    def _compute(b):
        def _group(g, _):
            gb = g * 16
            avec = zero16
            for el in range(16):
                e = gb + el
                t4 = jnp.maximum(src_v[b, e, pl.ds(32, 16)]
                                 + dst_v[b, e, pl.ds(32, 16)], 0.0)
                t5 = jnp.maximum(src_v[b, e, pl.ds(48, 16)]
                                 + dst_v[b, e, pl.ds(48, 16)], 0.0)
                w = t4 * aw0 + t5 * aw1
                for sh in (8, 4, 2, 1):
                    pad_v[el, pl.ds(0, 16)] = w
                    w = w + pad_v[el, pl.ds(sh, 16)]
                a = lax.squeeze(lax.slice(w, (0,), (1,)), (0,))
                avec = jnp.where(iota16 == el, a, avec)
            sg = 1.0 / (1.0 + jnp.exp(-(avec + a2v)))
            fw = ew_v[b, pl.ds(gb, 16)] * sg
            fw1_v[b, pl.ds(gb, 16)] = fw

            for el in range(16):
                e = gb + el
                fws = lax.squeeze(lax.slice(fw, (el,), (el + 1,)), (0,))
                fwb = jnp.full((16,), fws, jnp.float32)
                for d in range(HH // 16):
                    sv = src_v[b, e, pl.ds(16 * d, 16)]
                    dv = dst_v[b, e, pl.ds(16 * d, 16)]
                    out_v[b, e, pl.ds(16 * d, 16)] = (
                        jnp.maximum(sv + dv, 0.0) * fwb)
            return 0

        lax.fori_loop(0, K // 16, _group, 0)

    def _iter(i, _):
        for b in range(2):
            j = 2 * i + b

            @pl.when(i >= 1)
            def _():
                _drain_scatter(b)

            if b == 0:
                _drain_idx(1)
                _adj_idx(1)
                _issue_gather(1)
            else:
                @pl.when(i < NBH - 1)
                def _():
                    _drain_idx(0)
                    _adj_idx(0)
                    _issue_gather(0)

            _drain_gather(b)
            _compute(b)
            _issue_scatter(b)

            @pl.when(i < NBH - 1)
            def _():
                _issue_idx(j + 2, b)
        return 0

    lax.fori_loop(0, NBH, _iter, 0)
    _drain_scatter(0)
    _drain_scatter(1)
    plsc.subcore_barrier()

    cbase = s * CQ

    @pl.when(c == 0)
    def _():
        def _cp(k, _):
            pltpu.sync_copy(racc.at[pl.ds(cbase + k * 40, 40)],
                            r0_hbm.at[pl.ds(cbase + k * 40, 40)])
            return 0
        lax.fori_loop(0, CQ // 40, _cp, 0)
        pltpu.sync_copy(facc.at[pl.ds(cbase, CQ)],
                        f_hbm.at[pl.ds(cbase, CQ)])

        @pl.when(s == SC_SUB - 1)
        def _():
            def _cpt(k, _):
                pltpu.sync_copy(
                    racc.at[pl.ds(cbase + CQ + k * 40, 40)],
                    r0_hbm.at[pl.ds(cbase + CQ + k * 40, 40)])
                return 0
            lax.fori_loop(0, (N - SC_SUB * CQ) // 40, _cpt, 0)
            pltpu.sync_copy(facc.at[pl.ds(cbase + CQ, N - SC_SUB * CQ)],
                            f_hbm.at[pl.ds(cbase + CQ, N - SC_SUB * CQ)])

    @pl.when(c == 1)
    def _():
        def _cp(k, _):
            pltpu.sync_copy(racc.at[pl.ds(cbase + k * 40, 40)],
                            r1_hbm.at[pl.ds(cbase + k * 40, 40)])
            return 0
        lax.fori_loop(0, CQ // 40, _cp, 0)

        @pl.when(s == SC_SUB - 1)
        def _():
            def _cpt(k, _):
                pltpu.sync_copy(
                    racc.at[pl.ds(cbase + CQ + k * 40, 40)],
                    r1_hbm.at[pl.ds(cbase + CQ + k * 40, 40)])
                return 0
            lax.fori_loop(0, (N - SC_SUB * CQ) // 40, _cpt, 0)


def kernel(node_features, edge_index, edge_weights, params):
    p = params
    padi = jnp.zeros((EPAD - E,), jnp.int32)
    row = jnp.concatenate([edge_index[0], padi])
    col = jnp.concatenate([edge_index[1], padi])
    ew = jnp.concatenate([edge_weights, jnp.zeros((EPAD - E,), jnp.float32)])
    r2 = lambda b: b.reshape(1, -1)

    ap = [jnp.concatenate([p['l%d_att2_w' % i][:, 0],
                           jnp.full((16,), p['l%d_att2_b' % i][0],
                                    jnp.float32)]) for i in range(2)]

    x0, tsa0, tsb0, tda0, tdb0 = _embed_tables(
        node_features, p['embed_w'], r2(p['embed_b']),
        p['l0_msg1_w'], r2(p['l0_msg1_b']),
        p['l0_att1_w'], r2(p['l0_att1_b']))
    ts0 = jnp.concatenate([tsa0, tsb0], axis=0)
    td0 = jnp.concatenate([tda0, tdb0], axis=0)
    R00, R01, f0 = _sc_edges(ts0, td0, row, col, ew, ap[0])
    x1, tsa1, tsb1, tda1, tdb1 = _post_tables(
        x0, R00, R01, f0.reshape(N, 1),
        p['l0_msg2_w'], r2(p['l0_msg2_b']),
        p['l0_upd1_w'], r2(p['l0_upd1_b']),
        p['l0_upd2_w'], r2(p['l0_upd2_b']),
        p['l1_msg1_w'], r2(p['l1_msg1_b']),
        p['l1_att1_w'], r2(p['l1_att1_b']))
    ts1 = jnp.concatenate([tsa1, tsb1], axis=0)
    td1 = jnp.concatenate([tda1, tdb1], axis=0)
    R10, R11, f1 = _sc_edges(ts1, td1, row, col, ew, ap[1])
    out = _post_head(
        x1, R10, R11, f1.reshape(N, 1),
        p['l1_msg2_w'], r2(p['l1_msg2_b']),
        p['l1_upd1_w'], r2(p['l1_upd1_b']),
        p['l1_upd2_w'], r2(p['l1_upd2_b']),
        p['head_w'], r2(p['head_b']))
    return out[0, 0]
